# baseline (speedup 1.0000x reference)
import functools

import jax
import jax.numpy as jnp
from jax import lax
from jax.experimental import pallas as pl
from jax.experimental.pallas import tpu as pltpu
from jax.experimental.pallas import tpu_sc as plsc

D = 32
NC, NS = 2, 16
NW = NC * NS
NB, NSEQ = 16384, 50
CHUNK = 128
NCH = NSEQ * (NB // CHUNK)
CPW = NCH // NW
BTILES = NB // CHUNK
NBUF = 4

_mesh = plsc.VectorSubcoreMesh(core_axis_name="c", subcore_axis_name="s")


@functools.partial(
    pl.kernel,
    out_type=jax.ShapeDtypeStruct((NSEQ * 4 * BTILES, 8, CHUNK), jnp.float32),
    mesh=_mesh,
    scratch_types=(
        [pltpu.VMEM((CPW, CHUNK), jnp.int32)]
        + [pltpu.VMEM((CHUNK, D), jnp.float32)] * NBUF
        + [pltpu.VMEM((4, 8, CHUNK), jnp.float32)] * NBUF
        + [pltpu.SemaphoreType.DMA] * NBUF
        + [pltpu.SemaphoreType.DMA] * NBUF
    ),
    compiler_params=pltpu.CompilerParams(
        use_tc_tiling_on_sc=False, needs_layout_passes=False,
        disable_bounds_checks=True),
)
def _gather_kernel(idx_hbm, w_hbm, out_hbm, idx_v, *rest):
    rows = rest[:NBUF]
    tbufs = rest[NBUF:2 * NBUF]
    gsems = rest[2 * NBUF:3 * NBUF]
    osems = rest[3 * NBUF:]
    wid = lax.axis_index("s") * NC + lax.axis_index("c")
    cbase = wid * CPW
    base = jnp.arange(16, dtype=jnp.int32)
    rowvecs = [base + 16 * g for g in range(8)]

    pltpu.sync_copy(idx_hbm.at[pl.ds(cbase, CPW)], idx_v)

    for k in range(NBUF - 1):
        pltpu.async_copy(w_hbm.at[idx_v.at[k]], rows[k], gsems[k])

    @pl.loop(0, CPW, step=NBUF)
    def _(g):
        for k in range(NBUF):
            c = g + k
            ka = (k + NBUF - 1) % NBUF
            @pl.when(c + NBUF - 1 < CPW)
            def _():
                pltpu.async_copy(
                    w_hbm.at[idx_v.at[c + NBUF - 1]], rows[ka], gsems[ka])
            pltpu.make_async_copy(
                w_hbm.at[idx_v.at[c]], rows[k], gsems[k]).wait()
            @pl.when(c >= NBUF)
            def _():
                for ti in range(4):
                    pltpu.make_async_copy(
                        tbufs[k].at[ti], out_hbm.at[ti], osems[k]).wait()
            rk = rows[k]
            tk = tbufs[k]

            @plsc.parallel_loop(0, D, unroll=4)
            def _(d):
                col = jnp.full((16,), 0, jnp.int32) + d
                ti = d // 8
                dd = d - ti * 8
                for grp in range(8):
                    vec = plsc.load_gather(rk, [rowvecs[grp], col])
                    tk[ti, dd, pl.ds(16 * grp, 16)] = vec

            gg = cbase + c
            s = gg // BTILES
            tj = gg - s * BTILES
            rb = s * (4 * BTILES) + tj
            for ti in range(4):
                pltpu.async_copy(
                    tbufs[k].at[ti], out_hbm.at[rb + ti * BTILES], osems[k])

    for k in range(NBUF):
        for ti in range(4):
            pltpu.make_async_copy(
                tbufs[k].at[ti], out_hbm.at[ti], osems[k]).wait()


def kernel(idx, weight):
    idx2d = idx.T.astype(jnp.int32).reshape(NCH, CHUNK)
    out = _gather_kernel(idx2d, weight)
    out5 = out.reshape(NSEQ, 4, BTILES, 8, CHUNK)
    return out5.transpose(2, 4, 0, 1, 3).reshape(NB, NSEQ, D)


# speedup vs baseline: 1.4322x; 1.4322x over previous
import functools

import jax
import jax.numpy as jnp
from jax import lax
from jax.experimental import pallas as pl
from jax.experimental.pallas import tpu as pltpu
from jax.experimental.pallas import tpu_sc as plsc

D = 32
NC, NS = 2, 16
NW = NC * NS
NB, NSEQ = 16384, 50
CHUNK = 128
NCH = NSEQ * (NB // CHUNK)
CPW = NCH // NW
BTILES = NB // CHUNK
NBUF = 4

_mesh = plsc.VectorSubcoreMesh(core_axis_name="c", subcore_axis_name="s")


@functools.partial(
    pl.kernel,
    out_type=jax.ShapeDtypeStruct((NSEQ * 4 * BTILES, 8, CHUNK), jnp.float32),
    mesh=_mesh,
    scratch_types=(
        [pltpu.VMEM((CPW, CHUNK), jnp.int32)]
        + [pltpu.VMEM((CHUNK, D), jnp.float32)] * NBUF
        + [pltpu.VMEM((4, 8, CHUNK + 1), jnp.float32)] * NBUF
        + [pltpu.SemaphoreType.DMA] * NBUF
        + [pltpu.SemaphoreType.DMA] * NBUF
    ),
    compiler_params=pltpu.CompilerParams(
        use_tc_tiling_on_sc=False, needs_layout_passes=False,
        disable_bounds_checks=True),
)
def _gather_kernel(idx_hbm, w_hbm, out_hbm, idx_v, *rest):
    rows = rest[:NBUF]
    tbufs = rest[NBUF:2 * NBUF]
    gsems = rest[2 * NBUF:3 * NBUF]
    osems = rest[3 * NBUF:]
    wid = lax.axis_index("s") * NC + lax.axis_index("c")
    cbase = wid * CPW
    base = jnp.arange(16, dtype=jnp.int32)
    tihalf = [(base + 16 * h) // 8 for h in range(2)]
    ddvec = base % 8

    pltpu.sync_copy(idx_hbm.at[pl.ds(cbase, CPW)], idx_v)

    for k in range(NBUF - 1):
        pltpu.async_copy(w_hbm.at[idx_v.at[k]], rows[k], gsems[k])

    @pl.loop(0, CPW, step=NBUF)
    def _(g):
        for k in range(NBUF):
            c = g + k
            ka = (k + NBUF - 1) % NBUF
            @pl.when(c + NBUF - 1 < CPW)
            def _():
                pltpu.async_copy(
                    w_hbm.at[idx_v.at[c + NBUF - 1]], rows[ka], gsems[ka])
            pltpu.make_async_copy(
                w_hbm.at[idx_v.at[c]], rows[k], gsems[k]).wait()
            @pl.when(c >= NBUF)
            def _():
                for ti in range(4):
                    pltpu.make_async_copy(
                        tbufs[k].at[ti, :, pl.ds(0, CHUNK)],
                        out_hbm.at[ti], osems[k]).wait()
            rk = rows[k]
            tk = tbufs[k]

            @plsc.parallel_loop(0, CHUNK, unroll=4)
            def _(b):
                pos = jnp.full((16,), 0, jnp.int32) + b
                for half in range(2):
                    vec = rk[b, pl.ds(16 * half, 16)]
                    plsc.store_scatter(
                        tk, [tihalf[half], ddvec, pos], vec)

            gg = cbase + c
            s = gg // BTILES
            tj = gg - s * BTILES
            rb = s * (4 * BTILES) + tj
            for ti in range(4):
                pltpu.async_copy(
                    tbufs[k].at[ti, :, pl.ds(0, CHUNK)],
                    out_hbm.at[rb + ti * BTILES], osems[k])

    for k in range(NBUF):
        for ti in range(4):
            pltpu.make_async_copy(
                tbufs[k].at[ti, :, pl.ds(0, CHUNK)],
                out_hbm.at[ti], osems[k]).wait()


def kernel(idx, weight):
    idx2d = idx.T.astype(jnp.int32).reshape(NCH, CHUNK)
    out = _gather_kernel(idx2d, weight)
    out5 = out.reshape(NSEQ, 4, BTILES, 8, CHUNK)
    return out5.transpose(2, 4, 0, 1, 3).reshape(NB, NSEQ, D)
